# baseline (speedup 1.0000x reference)
import functools

import jax
import jax.numpy as jnp
from jax import lax
from jax.experimental import pallas as pl
from jax.experimental.pallas import tpu as pltpu
from jax.experimental.pallas import tpu_sc as plsc

VOCAB = 30522
D = 768
B = 128
S = 512
EPS = 1e-12

NW = 32
NSPLIT = 4
BSPLIT = B // NSPLIT
TOK_SPLIT = BSPLIT * S
TOK_PER_W = TOK_SPLIT // NW
CHUNK = 32
NCHUNK = TOK_PER_W // CHUNK
DW = D // 2


def _pack_chunk(src, dst):
    def row(r, _):
        for g in range(DW // 16):
            a = src[r, pl.ds(16 * g, 16)]
            b = src[r, pl.ds(DW + 16 * g, 16)]
            ai = lax.bitcast_convert_type(a, jnp.int32)
            bi = lax.bitcast_convert_type(b, jnp.int32)
            lo = lax.shift_right_logical(ai + jnp.int32(0x8000), 16)
            hi = (bi + jnp.int32(0x8000)) & jnp.int32(-65536)
            dst[r, pl.ds(16 * g, 16)] = lo | hi
        return 0
    lax.fori_loop(0, CHUNK, row, 0)


def _sc_gather(ids3, word_emb):
    mesh = plsc.VectorSubcoreMesh(core_axis_name="c", subcore_axis_name="s")

    @functools.partial(
        pl.kernel,
        mesh=mesh,
        out_type=jax.ShapeDtypeStruct((TOK_SPLIT, DW), jnp.int32),
        scratch_types=[
            pltpu.VMEM((NCHUNK, CHUNK), jnp.int32),
            pltpu.VMEM((CHUNK, D), jnp.float32),
            pltpu.VMEM((CHUNK, D), jnp.float32),
            pltpu.VMEM((CHUNK, DW), jnp.int32),
            pltpu.VMEM((CHUNK, DW), jnp.int32),
            pltpu.SemaphoreType.DMA,
            pltpu.SemaphoreType.DMA,
            pltpu.SemaphoreType.DMA,
            pltpu.SemaphoreType.DMA,
        ],
    )
    def k(ids_hbm, table_hbm, out_hbm, idx_v, buf0, buf1, bb0, bb1,
          g0, g1, o0, o1):
        cid = lax.axis_index("c")
        sid = lax.axis_index("s")
        wid = sid * 2 + cid
        base = wid * TOK_PER_W
        pltpu.sync_copy(ids_hbm.at[wid], idx_v)

        def gather(c, buf, sem):
            return pltpu.async_copy(table_hbm.at[idx_v.at[c]], buf, sem)

        def put(c, bb, sem):
            return pltpu.async_copy(
                bb, out_hbm.at[pl.ds(base + c * CHUNK, CHUNK)], sem)

        gather(0, buf0, g0)
        gather(1, buf1, g1)

        def body(i, _):
            c0 = 2 * i
            c1 = c0 + 1
            pltpu.make_async_copy(table_hbm.at[idx_v.at[c0]], buf0, g0).wait()

            @pl.when(i > 0)
            def _():
                pltpu.make_async_copy(
                    bb0, out_hbm.at[pl.ds(base + (c0 - 2) * CHUNK, CHUNK)],
                    o0).wait()
            _pack_chunk(buf0, bb0)

            @pl.when(i + 1 < NCHUNK // 2)
            def _():
                gather(c0 + 2, buf0, g0)
            put(c0, bb0, o0)

            pltpu.make_async_copy(table_hbm.at[idx_v.at[c1]], buf1, g1).wait()

            @pl.when(i > 0)
            def _():
                pltpu.make_async_copy(
                    bb1, out_hbm.at[pl.ds(base + (c1 - 2) * CHUNK, CHUNK)],
                    o1).wait()
            _pack_chunk(buf1, bb1)

            @pl.when(i + 1 < NCHUNK // 2)
            def _():
                gather(c1 + 2, buf1, g1)
            put(c1, bb1, o1)
            return 0

        lax.fori_loop(0, NCHUNK // 2, body, 0)
        last0 = NCHUNK - 2
        last1 = NCHUNK - 1
        pltpu.make_async_copy(
            bb0, out_hbm.at[pl.ds(base + last0 * CHUNK, CHUNK)], o0).wait()
        pltpu.make_async_copy(
            bb1, out_hbm.at[pl.ds(base + last1 * CHUNK, CHUNK)], o1).wait()

    return k(ids3, word_emb)


def _ln_body_first(w_ref, tt_ref, te_ref, pe_ref, g_ref, b_ref, o_ref):
    w = w_ref[0]
    xa = lax.bitcast_convert_type(w << 16, jnp.float32)
    xb = lax.bitcast_convert_type(w & jnp.int32(-65536), jnp.float32)
    t = tt_ref[0]
    e0 = te_ref[0:1, :]
    e1 = te_ref[1:2, :]
    td = e1 - e0
    pe = pe_ref[...]
    xa = xa + pe[:, :DW] + e0[:, :DW] + t * td[:, :DW]
    xb = xb + pe[:, DW:] + e0[:, DW:] + t * td[:, DW:]
    mean = (jnp.sum(xa, axis=-1, keepdims=True)
            + jnp.sum(xb, axis=-1, keepdims=True)) * (1.0 / D)
    xa = xa - mean
    xb = xb - mean
    var = (jnp.sum(xa * xa, axis=-1, keepdims=True)
           + jnp.sum(xb * xb, axis=-1, keepdims=True)) * (1.0 / D)
    inv = lax.rsqrt(var + EPS)
    g = g_ref[...]
    bb = b_ref[...]
    o_ref[0, :, :DW] = xa * inv * g[:, :DW] + bb[:, :DW]
    o_ref[0, :, DW:] = xb * inv * g[:, DW:] + bb[:, DW:]


def _ln_body(w_ref, tt_ref, te_ref, pe_ref, g_ref, b_ref, acc_ref, o_ref):
    del acc_ref
    _ln_body_first(w_ref, tt_ref, te_ref, pe_ref, g_ref, b_ref, o_ref)


def _tc_ln_part(w_rows, tt_col, type_emb, pos_emb, gamma, beta, prev, part):
    off = part * BSPLIT
    in_specs = [
        pl.BlockSpec((1, S, DW), lambda b: (b, 0, 0)),
        pl.BlockSpec((1, S, 1), lambda b, off=off: (off + b, 0, 0)),
        pl.BlockSpec((2, D), lambda b: (0, 0)),
        pl.BlockSpec((S, D), lambda b: (0, 0)),
        pl.BlockSpec((1, D), lambda b: (0, 0)),
        pl.BlockSpec((1, D), lambda b: (0, 0)),
    ]
    args = [w_rows, tt_col, type_emb, pos_emb, gamma, beta]
    if prev is None:
        body = _ln_body_first
        aliases = {}
    else:
        body = _ln_body
        in_specs.append(pl.BlockSpec(memory_space=pl.ANY))
        args.append(prev)
        aliases = {6: 0}
    return pl.pallas_call(
        body,
        grid=(BSPLIT,),
        in_specs=in_specs,
        out_specs=pl.BlockSpec((1, S, D), lambda b, off=off: (off + b, 0, 0)),
        out_shape=jax.ShapeDtypeStruct((B, S, D), jnp.float32),
        input_output_aliases=aliases,
    )(*args)


def kernel(input_ids, token_type_ids, word_emb, type_emb, pos_emb, gamma, beta):
    ids = input_ids.astype(jnp.int32).reshape(NSPLIT, NW, NCHUNK, CHUNK)
    tt_col = token_type_ids.astype(jnp.float32).reshape(B, S, 1)
    g2 = gamma.reshape(1, D)
    b2 = beta.reshape(1, D)
    ws = [_sc_gather(ids[i], word_emb) for i in range(NSPLIT)]
    out = None
    for i in range(NSPLIT):
        wp = ws[i].reshape(BSPLIT, S, DW)
        out = _tc_ln_part(wp, tt_col, type_emb, pos_emb, g2, b2, out, i)
    return out


# speedup vs baseline: 1.1317x; 1.1317x over previous
import functools

import jax
import jax.numpy as jnp
from jax import lax
from jax.experimental import pallas as pl
from jax.experimental.pallas import tpu as pltpu
from jax.experimental.pallas import tpu_sc as plsc

VOCAB = 30522
D = 768
B = 128
S = 512
EPS = 1e-12

NW = 32
NSPLIT = 4
BSPLIT = B // NSPLIT
TOK_SPLIT = BSPLIT * S
TOK_PER_W = TOK_SPLIT // NW
CHUNK = 64
NCHUNK = TOK_PER_W // CHUNK
DW = D // 2


def _pack_chunk(buf):
    def row(r, _):
        for g in range(DW // 16):
            a = buf[r, pl.ds(16 * g, 16)]
            b = buf[r, pl.ds(DW + 16 * g, 16)]
            lo = lax.shift_right_logical(a + jnp.int32(0x8000), 16)
            hi = (b + jnp.int32(0x8000)) & jnp.int32(-65536)
            buf[r, pl.ds(16 * g, 16)] = lo | hi
        return 0
    lax.fori_loop(0, CHUNK, row, 0)


def _sc_gather(ids3, word_emb_i32):
    mesh = plsc.VectorSubcoreMesh(core_axis_name="c", subcore_axis_name="s")

    @functools.partial(
        pl.kernel,
        mesh=mesh,
        out_type=jax.ShapeDtypeStruct((TOK_SPLIT, DW), jnp.int32),
        scratch_types=[
            pltpu.VMEM((NCHUNK, CHUNK), jnp.int32),
            pltpu.VMEM((CHUNK, D), jnp.int32),
            pltpu.VMEM((CHUNK, D), jnp.int32),
            pltpu.SemaphoreType.DMA,
            pltpu.SemaphoreType.DMA,
            pltpu.SemaphoreType.DMA,
            pltpu.SemaphoreType.DMA,
        ],
    )
    def k(ids_hbm, table_hbm, out_hbm, idx_v, buf0, buf1, g0, g1, o0, o1):
        cid = lax.axis_index("c")
        sid = lax.axis_index("s")
        wid = sid * 2 + cid
        base = wid * TOK_PER_W
        pltpu.sync_copy(ids_hbm.at[wid], idx_v)

        def gather(c, buf, sem):
            return pltpu.async_copy(table_hbm.at[idx_v.at[c]], buf, sem)

        def put(c, buf, sem):
            return pltpu.async_copy(
                buf.at[:, pl.ds(0, DW)],
                out_hbm.at[pl.ds(base + c * CHUNK, CHUNK)], sem)

        def put_wait(c, buf, sem):
            pltpu.make_async_copy(
                buf.at[:, pl.ds(0, DW)],
                out_hbm.at[pl.ds(base + c * CHUNK, CHUNK)], sem).wait()

        gather(0, buf0, g0)
        gather(1, buf1, g1)

        def body(i, _):
            c0 = 2 * i
            c1 = c0 + 1
            pltpu.make_async_copy(table_hbm.at[idx_v.at[c0]], buf0, g0).wait()
            _pack_chunk(buf0)
            put(c0, buf0, o0)

            pltpu.make_async_copy(table_hbm.at[idx_v.at[c1]], buf1, g1).wait()
            _pack_chunk(buf1)
            put(c1, buf1, o1)

            @pl.when(i + 1 < NCHUNK // 2)
            def _():
                put_wait(c0, buf0, o0)
                gather(c0 + 2, buf0, g0)
                put_wait(c1, buf1, o1)
                gather(c1 + 2, buf1, g1)
            return 0

        lax.fori_loop(0, NCHUNK // 2, body, 0)
        put_wait(NCHUNK - 2, buf0, o0)
        put_wait(NCHUNK - 1, buf1, o1)

    return k(ids3, word_emb_i32)


def _ln_body_first(w_ref, tt_ref, te_ref, pe_ref, g_ref, b_ref, o_ref):
    w = w_ref[0]
    xa = lax.bitcast_convert_type(w << 16, jnp.float32)
    xb = lax.bitcast_convert_type(w & jnp.int32(-65536), jnp.float32)
    t = tt_ref[0]
    e0 = te_ref[0:1, :]
    e1 = te_ref[1:2, :]
    td = e1 - e0
    pe = pe_ref[...]
    xa = xa + pe[:, :DW] + e0[:, :DW] + t * td[:, :DW]
    xb = xb + pe[:, DW:] + e0[:, DW:] + t * td[:, DW:]
    mean = (jnp.sum(xa, axis=-1, keepdims=True)
            + jnp.sum(xb, axis=-1, keepdims=True)) * (1.0 / D)
    xa = xa - mean
    xb = xb - mean
    var = (jnp.sum(xa * xa, axis=-1, keepdims=True)
           + jnp.sum(xb * xb, axis=-1, keepdims=True)) * (1.0 / D)
    inv = lax.rsqrt(var + EPS)
    g = g_ref[...]
    bb = b_ref[...]
    o_ref[0, :, :DW] = xa * inv * g[:, :DW] + bb[:, :DW]
    o_ref[0, :, DW:] = xb * inv * g[:, DW:] + bb[:, DW:]


def _ln_body(w_ref, tt_ref, te_ref, pe_ref, g_ref, b_ref, acc_ref, o_ref):
    del acc_ref
    _ln_body_first(w_ref, tt_ref, te_ref, pe_ref, g_ref, b_ref, o_ref)


def _tc_ln_part(w_rows, tt_col, type_emb, pos_emb, gamma, beta, prev, part):
    off = part * BSPLIT
    in_specs = [
        pl.BlockSpec((1, S, DW), lambda b: (b, 0, 0)),
        pl.BlockSpec((1, S, 1), lambda b, off=off: (off + b, 0, 0)),
        pl.BlockSpec((2, D), lambda b: (0, 0)),
        pl.BlockSpec((S, D), lambda b: (0, 0)),
        pl.BlockSpec((1, D), lambda b: (0, 0)),
        pl.BlockSpec((1, D), lambda b: (0, 0)),
    ]
    args = [w_rows, tt_col, type_emb, pos_emb, gamma, beta]
    if prev is None:
        body = _ln_body_first
        aliases = {}
    else:
        body = _ln_body
        in_specs.append(pl.BlockSpec(memory_space=pl.ANY))
        args.append(prev)
        aliases = {6: 0}
    return pl.pallas_call(
        body,
        grid=(BSPLIT,),
        in_specs=in_specs,
        out_specs=pl.BlockSpec((1, S, D), lambda b, off=off: (off + b, 0, 0)),
        out_shape=jax.ShapeDtypeStruct((B, S, D), jnp.float32),
        input_output_aliases=aliases,
    )(*args)


def kernel(input_ids, token_type_ids, word_emb, type_emb, pos_emb, gamma, beta):
    ids = input_ids.astype(jnp.int32).reshape(NSPLIT, NW, NCHUNK, CHUNK)
    tt_col = token_type_ids.astype(jnp.float32).reshape(B, S, 1)
    g2 = gamma.reshape(1, D)
    b2 = beta.reshape(1, D)
    wi32 = lax.bitcast_convert_type(word_emb, jnp.int32)
    ws = [_sc_gather(ids[i], wi32) for i in range(NSPLIT)]
    out = None
    for i in range(NSPLIT):
        wp = ws[i].reshape(BSPLIT, S, DW)
        out = _tc_ln_part(wp, tt_col, type_emb, pos_emb, g2, b2, out, i)
    return out
